# baseline (speedup 1.0000x reference)
import functools

import jax
import jax.numpy as jnp
from jax.experimental import pallas as pl

K_NN = 10
_INF = float("inf")


def _extract_naive(d2, k_nn):
    tr, m = d2.shape
    iota = jax.lax.broadcasted_iota(jnp.int32, (tr, m), 1)
    cur = d2
    vals, idxs = [], []
    for _ in range(k_nn):
        mn = jnp.min(cur, axis=1, keepdims=True)
        am = jnp.min(jnp.where(cur == mn, iota, m), axis=1, keepdims=True)
        vals.append(mn)
        idxs.append(am)
        cur = jnp.where(iota == am, _INF, cur)
    return jnp.concatenate(idxs, axis=1), jnp.concatenate(vals, axis=1)


def _extract_fast(d2, k_nn, depth_a):
    tr, m = d2.shape
    b = m // depth_a
    d3 = d2.reshape(tr, depth_a, b)
    ia = jax.lax.broadcasted_iota(jnp.int32, (tr, depth_a, b), 1)
    ib = jax.lax.broadcasted_iota(jnp.int32, (tr, depth_a, b), 2)
    iota3 = ia * b + ib

    cur = d3
    cvals, cidxs = [], []
    for r in range(3):
        mv = jnp.min(cur, axis=1)
        mi = jnp.min(jnp.where(cur == mv[:, None, :], iota3, m), axis=1)
        cvals.append(mv)
        cidxs.append(mi)
        if r < 2:
            cur = jnp.where(iota3 == mi[:, None, :], _INF, cur)
    c1, c2, c3 = cvals
    i1, i2, i3 = cidxs

    vals, idxs = [], []
    for _ in range(k_nn):
        mn = jnp.min(c1, axis=1, keepdims=True)
        oi = jnp.min(jnp.where(c1 == mn, i1, m), axis=1, keepdims=True)
        vals.append(mn)
        idxs.append(oi)
        hit = (c1 == mn) & (i1 == oi)
        c1 = jnp.where(hit, c2, c1)
        i1 = jnp.where(hit, i2, i1)
        c2 = jnp.where(hit, c3, c2)
        i2 = jnp.where(hit, i3, i2)
        c3 = jnp.where(hit, _INF, c3)
        i3 = jnp.where(hit, m, i3)

    classes = [oi % b for oi in idxs]
    trip = jnp.zeros_like(classes[0], dtype=jnp.bool_)
    for k in range(k_nn):
        cnt = None
        for j in range(k_nn):
            if j == k:
                continue
            e = (classes[j] == classes[k]).astype(jnp.int32)
            cnt = e if cnt is None else cnt + e
        trip = trip | (cnt >= 2)
    return (jnp.concatenate(idxs, axis=1), jnp.concatenate(vals, axis=1),
            jnp.any(trip))


def _knn_block(ksq_ref, qsq_ref, keys_ref, q_ref, idx_ref, dist_ref, *, k_nn):
    def compute_d2():
        dot = jax.lax.dot_general(
            keys_ref[...], q_ref[...],
            dimension_numbers=(((1,), (1,)), ((), ())),
            preferred_element_type=jnp.float32,
            precision=jax.lax.Precision.DEFAULT,
        )
        d2 = (ksq_ref[...] + qsq_ref[...]) - 2.0 * dot
        return jnp.maximum(d2, 0.0)

    fi, fv, fallback = _extract_fast(compute_d2(), k_nn, depth_a=16)
    idx, dist = jax.lax.cond(
        fallback,
        lambda: _extract_naive(compute_d2(), k_nn),
        lambda: (fi, fv),
    )
    idx_ref[...] = idx
    dist_ref[...] = dist


@functools.partial(jax.jit, static_argnames=())
def kernel(keys, queries):
    n, d = keys.shape
    m, _ = queries.shape
    ksq = jnp.sum(keys * keys, axis=1, keepdims=True)
    qsq = jnp.sum(queries * queries, axis=1, keepdims=True)
    qsq_row = qsq.T

    tile_r = 64 if n % 64 == 0 else n
    grid = (n // tile_r,)

    idx_out, dist_out = pl.pallas_call(
        functools.partial(_knn_block, k_nn=K_NN),
        grid=grid,
        in_specs=[
            pl.BlockSpec((tile_r, 1), lambda i: (i, 0)),
            pl.BlockSpec((1, m), lambda i: (0, 0)),
            pl.BlockSpec((tile_r, d), lambda i: (i, 0)),
            pl.BlockSpec((m, d), lambda i: (0, 0)),
        ],
        out_specs=[
            pl.BlockSpec((tile_r, K_NN), lambda i: (i, 0)),
            pl.BlockSpec((tile_r, K_NN), lambda i: (i, 0)),
        ],
        out_shape=[
            jax.ShapeDtypeStruct((n, K_NN), jnp.int32),
            jax.ShapeDtypeStruct((n, K_NN), jnp.float32),
        ],
    )(ksq, qsq_row, keys, queries)
    return idx_out, dist_out


# speedup vs baseline: 2.5398x; 2.5398x over previous
import functools

import jax
import jax.numpy as jnp
from jax.experimental import pallas as pl

K_NN = 10
_INF = float("inf")
_A = 16


def _extract_naive(d2, k_nn):
    tr, m = d2.shape
    iota = jax.lax.broadcasted_iota(jnp.int32, (tr, m), 1)
    cur = d2
    vals, idxs = [], []
    for _ in range(k_nn):
        mn = jnp.min(cur, axis=1, keepdims=True)
        am = jnp.min(jnp.where(cur == mn, iota, m), axis=1, keepdims=True)
        vals.append(mn)
        idxs.append(am)
        cur = jnp.where(iota == am, _INF, cur)
    return jnp.concatenate(idxs, axis=1), jnp.concatenate(vals, axis=1)


def _class_top3(d2, m):
    tr = d2.shape[0]
    b = m // _A
    ib = jax.lax.broadcasted_iota(jnp.int32, (tr, b), 1)
    slices = [d2[:, a * b:(a + 1) * b] for a in range(_A)]
    cols = [ib + a * b for a in range(_A)]

    out = []
    for rnd in range(3):
        mv = slices[0]
        for a in range(1, _A):
            mv = jnp.minimum(mv, slices[a])
        mi = None
        for a in range(_A):
            cand = jnp.where(slices[a] == mv, cols[a], m)
            mi = cand if mi is None else jnp.minimum(mi, cand)
        out.append((mv, mi))
        if rnd < 2:
            slices = [jnp.where(cols[a] == mi, _INF, slices[a])
                      for a in range(_A)]
    return out


def _extract_fast(d2, k_nn):
    tr, m = d2.shape
    b = m // _A
    (c1, i1), (c2, i2), (c3, i3) = _class_top3(d2, m)

    wcnt = jnp.zeros((tr, b), dtype=jnp.int32)
    vals, idxs = [], []
    for _ in range(k_nn):
        mn = jnp.min(c1, axis=1, keepdims=True)
        oi = jnp.min(jnp.where(c1 == mn, i1, m), axis=1, keepdims=True)
        vals.append(mn)
        idxs.append(oi)
        hit = (c1 == mn) & (i1 == oi)
        wcnt = wcnt + hit.astype(jnp.int32)
        c1 = jnp.where(hit, c2, c1)
        i1 = jnp.where(hit, i2, i1)
        c2 = jnp.where(hit, c3, c2)
        i2 = jnp.where(hit, i3, i2)
        c3 = jnp.where(hit, _INF, c3)
        i3 = jnp.where(hit, m, i3)

    fallback = jnp.max(wcnt) >= 3
    return (jnp.concatenate(idxs, axis=1), jnp.concatenate(vals, axis=1),
            fallback)


def _knn_block(ksq_ref, qsq_ref, keys_ref, q_ref, idx_ref, dist_ref, *, k_nn):
    def compute_d2():
        dot = jax.lax.dot_general(
            keys_ref[...], q_ref[...],
            dimension_numbers=(((1,), (1,)), ((), ())),
            preferred_element_type=jnp.float32,
            precision=jax.lax.Precision.DEFAULT,
        )
        d2 = (ksq_ref[...] + qsq_ref[...]) - 2.0 * dot
        return jnp.maximum(d2, 0.0)

    fi, fv, fallback = _extract_fast(compute_d2(), k_nn)
    idx, dist = jax.lax.cond(
        fallback,
        lambda: _extract_naive(compute_d2(), k_nn),
        lambda: (fi, fv),
    )
    idx_ref[...] = idx
    dist_ref[...] = dist


@functools.partial(jax.jit, static_argnames=())
def kernel(keys, queries):
    n, d = keys.shape
    m, _ = queries.shape
    ksq = jnp.sum(keys * keys, axis=1, keepdims=True)
    qsq = jnp.sum(queries * queries, axis=1, keepdims=True)
    qsq_row = qsq.T

    tile_r = 128 if n % 128 == 0 else n
    grid = (n // tile_r,)

    idx_out, dist_out = pl.pallas_call(
        functools.partial(_knn_block, k_nn=K_NN),
        grid=grid,
        in_specs=[
            pl.BlockSpec((tile_r, 1), lambda i: (i, 0)),
            pl.BlockSpec((1, m), lambda i: (0, 0)),
            pl.BlockSpec((tile_r, d), lambda i: (i, 0)),
            pl.BlockSpec((m, d), lambda i: (0, 0)),
        ],
        out_specs=[
            pl.BlockSpec((tile_r, K_NN), lambda i: (i, 0)),
            pl.BlockSpec((tile_r, K_NN), lambda i: (i, 0)),
        ],
        out_shape=[
            jax.ShapeDtypeStruct((n, K_NN), jnp.int32),
            jax.ShapeDtypeStruct((n, K_NN), jnp.float32),
        ],
    )(ksq, qsq_row, keys, queries)
    return idx_out, dist_out


# speedup vs baseline: 5.7602x; 2.2680x over previous
import functools

import jax
import jax.numpy as jnp
from jax.experimental import pallas as pl

K_NN = 10
_INF = float("inf")
_A = 32
_DEPTH = 4


def _compute_d2(ksq_ref, qsq_ref, keys_ref, q_ref):
    dot = jax.lax.dot_general(
        keys_ref[...], q_ref[...],
        dimension_numbers=(((1,), (1,)), ((), ())),
        preferred_element_type=jnp.float32,
        precision=jax.lax.Precision.DEFAULT,
    )
    d2 = (ksq_ref[...] + qsq_ref[...]) - 2.0 * dot
    return jnp.maximum(d2, 0.0)


def _naive_block(ksq_ref, qsq_ref, keys_ref, q_ref, idx_ref, dist_ref, *, k_nn):
    d2 = _compute_d2(ksq_ref, qsq_ref, keys_ref, q_ref)
    tr, m = d2.shape
    iota = jax.lax.broadcasted_iota(jnp.int32, (tr, m), 1)
    cur = d2
    vals, idxs = [], []
    for _ in range(k_nn):
        mn = jnp.min(cur, axis=1, keepdims=True)
        am = jnp.min(jnp.where(cur == mn, iota, m), axis=1, keepdims=True)
        vals.append(mn)
        idxs.append(am)
        cur = jnp.where(iota == am, _INF, cur)
    idx_ref[...] = jnp.concatenate(idxs, axis=1)
    dist_ref[...] = jnp.concatenate(vals, axis=1)


def _fast_block(ksq_ref, qsq_ref, keys_ref, q_ref, idx_ref, dist_ref, w_ref,
                *, k_nn):
    d2 = _compute_d2(ksq_ref, qsq_ref, keys_ref, q_ref)
    tr, m = d2.shape
    b = m // _A

    ib = jax.lax.broadcasted_iota(jnp.int32, (tr, b), 1)
    slices = [d2[:, a * b:(a + 1) * b] for a in range(_A)]
    cols = [ib + a * b for a in range(_A)]
    cv, ci = [], []
    for rnd in range(_DEPTH):
        mv = slices[0]
        for a in range(1, _A):
            mv = jnp.minimum(mv, slices[a])
        mi = None
        for a in range(_A):
            cand = jnp.where(slices[a] == mv, cols[a], m)
            mi = cand if mi is None else jnp.minimum(mi, cand)
        cv.append(mv)
        ci.append(mi)
        if rnd < _DEPTH - 1:
            slices = [jnp.where(cols[a] == mi, _INF, slices[a])
                      for a in range(_A)]

    wcnt = jnp.zeros((tr, b), dtype=jnp.int32)
    vals, idxs = [], []
    for _ in range(k_nn):
        mn = jnp.min(cv[0], axis=1, keepdims=True)
        oi = jnp.min(jnp.where(cv[0] == mn, ci[0], m), axis=1,
                     keepdims=True)
        vals.append(mn)
        idxs.append(oi)
        hit = (cv[0] == mn) & (ci[0] == oi)
        wcnt = wcnt + hit.astype(jnp.int32)
        for dpt in range(_DEPTH - 1):
            cv[dpt] = jnp.where(hit, cv[dpt + 1], cv[dpt])
            ci[dpt] = jnp.where(hit, ci[dpt + 1], ci[dpt])
        cv[_DEPTH - 1] = jnp.where(hit, _INF, cv[_DEPTH - 1])
        ci[_DEPTH - 1] = jnp.where(hit, m, ci[_DEPTH - 1])

    idx_ref[...] = jnp.concatenate(idxs, axis=1)
    dist_ref[...] = jnp.concatenate(vals, axis=1)
    w_ref[...] = jnp.max(wcnt, axis=1, keepdims=True)


def _row_specs(tile_r, m, d):
    return [
        pl.BlockSpec((tile_r, 1), lambda i: (i, 0)),
        pl.BlockSpec((1, m), lambda i: (0, 0)),
        pl.BlockSpec((tile_r, d), lambda i: (i, 0)),
        pl.BlockSpec((m, d), lambda i: (0, 0)),
    ]


@functools.partial(jax.jit, static_argnames=())
def kernel(keys, queries):
    n, d = keys.shape
    m, _ = queries.shape
    ksq = jnp.sum(keys * keys, axis=1, keepdims=True)
    qsq = jnp.sum(queries * queries, axis=1, keepdims=True)
    qsq_row = qsq.T
    tile_r = 128 if n % 128 == 0 else n
    grid = (n // tile_r,)
    args = (ksq, qsq_row, keys, queries)

    fi, fv, w = pl.pallas_call(
        functools.partial(_fast_block, k_nn=K_NN),
        grid=grid,
        in_specs=_row_specs(tile_r, m, d),
        out_specs=[
            pl.BlockSpec((tile_r, K_NN), lambda i: (i, 0)),
            pl.BlockSpec((tile_r, K_NN), lambda i: (i, 0)),
            pl.BlockSpec((tile_r, 1), lambda i: (i, 0)),
        ],
        out_shape=[
            jax.ShapeDtypeStruct((n, K_NN), jnp.int32),
            jax.ShapeDtypeStruct((n, K_NN), jnp.float32),
            jax.ShapeDtypeStruct((n, 1), jnp.int32),
        ],
    )(*args)

    def naive_all():
        return tuple(pl.pallas_call(
            functools.partial(_naive_block, k_nn=K_NN),
            grid=grid,
            in_specs=_row_specs(tile_r, m, d),
            out_specs=[
                pl.BlockSpec((tile_r, K_NN), lambda i: (i, 0)),
                pl.BlockSpec((tile_r, K_NN), lambda i: (i, 0)),
            ],
            out_shape=[
                jax.ShapeDtypeStruct((n, K_NN), jnp.int32),
                jax.ShapeDtypeStruct((n, K_NN), jnp.float32),
            ],
        )(*args))

    idx_out, dist_out = jax.lax.cond(
        jnp.max(w) >= _DEPTH,
        naive_all,
        lambda: (fi, fv),
    )
    return idx_out, dist_out


# speedup vs baseline: 6.4989x; 1.1282x over previous
import functools

import jax
import jax.numpy as jnp
from jax.experimental import pallas as pl

K_NN = 10
_INF = float("inf")
_A = 32
_DEPTH = 4


def _compute_d2(ksq_ref, qsq_ref, keys_ref, q_ref):
    dot = jax.lax.dot_general(
        keys_ref[...], q_ref[...],
        dimension_numbers=(((1,), (1,)), ((), ())),
        preferred_element_type=jnp.float32,
        precision=jax.lax.Precision.DEFAULT,
    )
    d2 = (ksq_ref[...] + qsq_ref[...]) - 2.0 * dot
    return jnp.maximum(d2, 0.0)


def _naive_block(ksq_ref, qsq_ref, keys_ref, q_ref, idx_ref, dist_ref, *, k_nn):
    d2 = _compute_d2(ksq_ref, qsq_ref, keys_ref, q_ref)
    tr, m = d2.shape
    iota = jax.lax.broadcasted_iota(jnp.int32, (tr, m), 1)
    cur = d2
    vals, idxs = [], []
    for _ in range(k_nn):
        mn = jnp.min(cur, axis=1, keepdims=True)
        am = jnp.min(jnp.where(cur == mn, iota, m), axis=1, keepdims=True)
        vals.append(mn)
        idxs.append(am)
        cur = jnp.where(iota == am, _INF, cur)
    idx_ref[...] = jnp.concatenate(idxs, axis=1)
    dist_ref[...] = jnp.concatenate(vals, axis=1)


def _ce(a, b):
    sw = b[0] < a[0]
    lo = (jnp.minimum(a[0], b[0]), jnp.where(sw, b[1], a[1]))
    hi = (jnp.maximum(a[0], b[0]), jnp.where(sw, a[1], b[1]))
    return lo, hi


def _ce_lo(a, b):
    sw = b[0] < a[0]
    return (jnp.minimum(a[0], b[0]), jnp.where(sw, b[1], a[1]))


def _lex_ce(a, b):
    sw = (b[0] < a[0]) | ((b[0] == a[0]) & (b[1] < a[1]))
    lo = (jnp.where(sw, b[0], a[0]), jnp.where(sw, b[1], a[1]))
    hi = (jnp.where(sw, a[0], b[0]), jnp.where(sw, a[1], b[1]))
    return lo, hi


def _merge22(x, y):
    a0, a1 = _ce(x[0], y[0])
    b0, b1 = _ce(x[1], y[1])
    m1, m2 = _lex_ce(a1, b0)
    return [a0, m1, m2, b1]


def _merge44(x, y):
    z = [_ce_lo(x[i], y[3 - i]) for i in range(4)]
    a0, a2 = _lex_ce(z[0], z[2])
    a1, a3 = _lex_ce(z[1], z[3])
    r0, r1 = _lex_ce(a0, a1)
    r2, r3 = _lex_ce(a2, a3)
    return [r0, r1, r2, r3]


def _fast_block(ksq_ref, qsq_ref, keys_ref, q_ref, idx_ref, dist_ref, w_ref,
                *, k_nn):
    d2 = _compute_d2(ksq_ref, qsq_ref, keys_ref, q_ref)
    tr, m = d2.shape
    b = m // _A

    ib = jax.lax.broadcasted_iota(jnp.int32, (tr, b), 1)
    pairs = [(d2[:, a * b:(a + 1) * b], ib + a * b) for a in range(_A)]
    l2 = [list(_ce(pairs[2 * i], pairs[2 * i + 1]))
          for i in range(_A // 2)]
    l4 = [_merge22(l2[2 * i], l2[2 * i + 1]) for i in range(_A // 4)]
    while len(l4) > 1:
        l4 = [_merge44(l4[2 * i], l4[2 * i + 1]) for i in range(len(l4) // 2)]
    cv = [t[0] for t in l4[0]]
    ci = [t[1] for t in l4[0]]

    wcnt = jnp.zeros((tr, b), dtype=jnp.int32)
    vals, idxs = [], []
    for _ in range(k_nn):
        mn = jnp.min(cv[0], axis=1, keepdims=True)
        oi = jnp.min(jnp.where(cv[0] == mn, ci[0], m), axis=1,
                     keepdims=True)
        vals.append(mn)
        idxs.append(oi)
        hit = (cv[0] == mn) & (ci[0] == oi)
        wcnt = wcnt + hit.astype(jnp.int32)
        for dpt in range(_DEPTH - 1):
            cv[dpt] = jnp.where(hit, cv[dpt + 1], cv[dpt])
            ci[dpt] = jnp.where(hit, ci[dpt + 1], ci[dpt])
        cv[_DEPTH - 1] = jnp.where(hit, _INF, cv[_DEPTH - 1])
        ci[_DEPTH - 1] = jnp.where(hit, m, ci[_DEPTH - 1])

    idx_ref[...] = jnp.concatenate(idxs, axis=1)
    dist_ref[...] = jnp.concatenate(vals, axis=1)
    w_ref[...] = jnp.max(wcnt, axis=1, keepdims=True)


def _row_specs(tile_r, m, d):
    return [
        pl.BlockSpec((tile_r, 1), lambda i: (i, 0)),
        pl.BlockSpec((1, m), lambda i: (0, 0)),
        pl.BlockSpec((tile_r, d), lambda i: (i, 0)),
        pl.BlockSpec((m, d), lambda i: (0, 0)),
    ]


@functools.partial(jax.jit, static_argnames=())
def kernel(keys, queries):
    n, d = keys.shape
    m, _ = queries.shape
    ksq = jnp.sum(keys * keys, axis=1, keepdims=True)
    qsq = jnp.sum(queries * queries, axis=1, keepdims=True)
    qsq_row = qsq.T
    tile_r = 128 if n % 128 == 0 else n
    grid = (n // tile_r,)
    args = (ksq, qsq_row, keys, queries)

    fi, fv, w = pl.pallas_call(
        functools.partial(_fast_block, k_nn=K_NN),
        grid=grid,
        in_specs=_row_specs(tile_r, m, d),
        out_specs=[
            pl.BlockSpec((tile_r, K_NN), lambda i: (i, 0)),
            pl.BlockSpec((tile_r, K_NN), lambda i: (i, 0)),
            pl.BlockSpec((tile_r, 1), lambda i: (i, 0)),
        ],
        out_shape=[
            jax.ShapeDtypeStruct((n, K_NN), jnp.int32),
            jax.ShapeDtypeStruct((n, K_NN), jnp.float32),
            jax.ShapeDtypeStruct((n, 1), jnp.int32),
        ],
    )(*args)

    def naive_all():
        return tuple(pl.pallas_call(
            functools.partial(_naive_block, k_nn=K_NN),
            grid=grid,
            in_specs=_row_specs(tile_r, m, d),
            out_specs=[
                pl.BlockSpec((tile_r, K_NN), lambda i: (i, 0)),
                pl.BlockSpec((tile_r, K_NN), lambda i: (i, 0)),
            ],
            out_shape=[
                jax.ShapeDtypeStruct((n, K_NN), jnp.int32),
                jax.ShapeDtypeStruct((n, K_NN), jnp.float32),
            ],
        )(*args))

    idx_out, dist_out = jax.lax.cond(
        jnp.max(w) >= _DEPTH,
        naive_all,
        lambda: (fi, fv),
    )
    return idx_out, dist_out
